# baseline (speedup 1.0000x reference)
import functools

import jax
import jax.numpy as jnp
from jax import lax
from jax.experimental import pallas as pl
from jax.experimental.pallas import tpu as pltpu
from jax.experimental.pallas import tpu_sc as plsc

NUM_ITEMS = 1_000_000
TBL = 1 << 20
TR, TC_ = TBL // 128, 128
ROWS, COLS = 16384, 200
CPAD = 256
NC, NS = 2, 16
NW = NC * NS
ROWS_PER_W = ROWS // NW
RW = 64
NWIN = ROWS_PER_W // RW
WIN = RW * CPAD
N2 = ROWS * CPAD
TSTG = 1_001_472

GB = 8
BR = ROWS // GB
BG = BR // 8


def _table_body(x_ref, o_ref):
    x = x_ref[...]
    prev_rows = jnp.concatenate(
        [jnp.full((1, TC_), -1e30, jnp.float32), x[:-1, :]], axis=0
    )
    col = lax.broadcasted_iota(jnp.int32, (TR, TC_), 1)
    row = lax.broadcasted_iota(jnp.int32, (TR, TC_), 0)
    xsel = jnp.where(col == TC_ - 1, prev_rows, x)
    shifted = pltpu.roll(xsel, 1, axis=1)
    tbl = jax.nn.sigmoid(8.0 * shifted)
    o_ref[...] = jnp.where((row == 0) & (col == 0), 0.0, tbl).reshape(TBL)


def _pre_body(x_ref, o_ref):
    x3 = x_ref[...].reshape(BG, 8, COLS)
    o_ref[:, 0] = x3[:, :, :128]
    o_ref[:, 1] = jnp.concatenate(
        [x3[:, :, 128:], jnp.zeros((BG, 8, CPAD - COLS), jnp.int32)], axis=2
    )


def _post_body(x_ref, o_ref):
    a = x_ref[:, 0]
    b = x_ref[:, 1][:, :, : COLS - 128]
    o_ref[...] = jnp.concatenate([a, b], axis=2).reshape(BR, COLS)


@functools.partial(
    pl.kernel,
    out_type=jax.ShapeDtypeStruct((N2,), jnp.float32),
    mesh=plsc.VectorSubcoreMesh(core_axis_name="c", subcore_axis_name="s"),
    scratch_types=[
        pltpu.VMEM_SHARED((TSTG,), jnp.float32),
        pltpu.VMEM((WIN,), jnp.int32),
        pltpu.VMEM((WIN,), jnp.int32),
        pltpu.VMEM((WIN,), jnp.float32),
        pltpu.VMEM((WIN,), jnp.float32),
        pltpu.SemaphoreType.DMA,
        pltpu.SemaphoreType.DMA,
        pltpu.SemaphoreType.DMA,
        pltpu.SemaphoreType.DMA,
        pltpu.SemaphoreType.DMA,
        pltpu.SemaphoreType.DMA,
    ],
)
def _gather(table_hbm, idx_hbm, out_hbm, tbl_sp,
            idx_v0, idx_v1, out_v0, out_v1,
            in_s0, in_s1, g_s0, g_s1, st_s0, st_s1):
    c = lax.axis_index("c")
    s = lax.axis_index("s")
    wid = s * NC + c
    base = wid * (N2 // NW)

    idx_v = (idx_v0, idx_v1)
    out_v = (out_v0, out_v1)
    in_s = (in_s0, in_s1)

    in_h = [None, None]
    st_h = [None, None]
    in_h[0] = pltpu.async_copy(idx_hbm.at[pl.ds(base, WIN)], idx_v[0], in_s[0])

    seg = TSTG // NS
    pltpu.sync_copy(table_hbm.at[pl.ds(s * seg, seg)], tbl_sp.at[pl.ds(s * seg, seg)])
    plsc.subcore_barrier()

    for w in range(NWIN):
        b = w & 1
        if w + 1 < NWIN:
            off = base + (w + 1) * WIN
            in_h[1 - b] = pltpu.async_copy(
                idx_hbm.at[pl.ds(off, WIN)], idx_v[1 - b], in_s[1 - b]
            )
        in_h[b].wait()
        if w >= 2:
            st_h[b].wait()
        pltpu.async_copy(tbl_sp.at[idx_v[b]], out_v[b], (g_s0, g_s1)[b]).wait()
        st_h[b] = pltpu.async_copy(
            out_v[b], out_hbm.at[pl.ds(base + w * WIN, WIN)], (st_s0, st_s1)[b]
        )
    st_h[0].wait()
    st_h[1].wait()


def kernel(inputs, item_rating_logits):
    pad = jnp.full((TBL - NUM_ITEMS + 1,), -1e30, jnp.float32)
    x = jnp.concatenate([item_rating_logits, pad]).reshape(TR, TC_)
    table = pl.pallas_call(
        _table_body,
        out_shape=jax.ShapeDtypeStruct((TBL,), jnp.float32),
    )(x)
    idxt = pl.pallas_call(
        _pre_body,
        grid=(GB,),
        in_specs=[pl.BlockSpec((BR, COLS), lambda i: (i, 0))],
        out_specs=pl.BlockSpec((BG, 2, 8, 128), lambda i: (i, 0, 0, 0)),
        out_shape=jax.ShapeDtypeStruct((GB * BG, 2, 8, 128), jnp.int32),
    )(inputs[0])
    outt = _gather(table, idxt.reshape(N2))
    out = pl.pallas_call(
        _post_body,
        grid=(GB,),
        in_specs=[pl.BlockSpec((BG, 2, 8, 128), lambda i: (i, 0, 0, 0))],
        out_specs=pl.BlockSpec((BR, COLS), lambda i: (i, 0)),
        out_shape=jax.ShapeDtypeStruct((ROWS, COLS), jnp.float32),
    )(outt.reshape(GB * BG, 2, 8, 128))
    return out


# speedup vs baseline: 2.8703x; 2.8703x over previous
import functools

import jax
import jax.numpy as jnp
from jax import lax
from jax.experimental import pallas as pl
from jax.experimental.pallas import tpu as pltpu
from jax.experimental.pallas import tpu_sc as plsc

NUM_ITEMS = 1_000_000
TBL = 1 << 20
TR, TC_ = TBL // 128, 128
ROWS, COLS = 16384, 200
CPAD = 256
NC, NS = 2, 16
NW = NC * NS
ROWS_PER_W = ROWS // NW
RW = 64
NWIN = ROWS_PER_W // RW
WIN = RW * CPAD
N2 = ROWS * CPAD
TSTG = 1_001_472

GB = 8
BR = ROWS // GB
BG = BR // 8
NPL = CPAD - COLS


def _table_body(x_ref, o_ref):
    x = x_ref[...]
    prev_rows = jnp.concatenate(
        [jnp.full((1, TC_), -1e30, jnp.float32), x[:-1, :]], axis=0
    )
    col = lax.broadcasted_iota(jnp.int32, (TR, TC_), 1)
    row = lax.broadcasted_iota(jnp.int32, (TR, TC_), 0)
    xsel = jnp.where(col == TC_ - 1, prev_rows, x)
    shifted = pltpu.roll(xsel, 1, axis=1)
    tbl = jax.nn.sigmoid(8.0 * shifted)
    o_ref[...] = jnp.where((row == 0) & (col == 0), 0.0, tbl).reshape(TBL)


def _pre_body(x_ref, o_ref):
    pid = pl.program_id(0)
    x3 = x_ref[...].reshape(BG, 8, COLS)
    g = lax.broadcasted_iota(jnp.int32, (BG, 8, NPL), 0)
    r = lax.broadcasted_iota(jnp.int32, (BG, 8, NPL), 1)
    l = lax.broadcasted_iota(jnp.int32, (BG, 8, NPL), 2)
    pos = ((pid * BG + g) * 8 + r) * NPL + l
    dummy = jnp.bitwise_and(pos * 40503, (1 << 19) - 1)
    o_ref[:, 0] = x3[:, :, :128]
    o_ref[:, 1] = jnp.concatenate([x3[:, :, 128:], dummy], axis=2)


def _post_body(x_ref, o_ref):
    a = x_ref[:, 0]
    b = x_ref[:, 1][:, :, : COLS - 128]
    o_ref[...] = jnp.concatenate([a, b], axis=2).reshape(BR, COLS)


@functools.partial(
    pl.kernel,
    out_type=jax.ShapeDtypeStruct((N2,), jnp.float32),
    mesh=plsc.VectorSubcoreMesh(core_axis_name="c", subcore_axis_name="s"),
    scratch_types=[
        pltpu.VMEM_SHARED((TSTG,), jnp.float32),
        pltpu.VMEM((WIN,), jnp.int32),
        pltpu.VMEM((WIN,), jnp.int32),
        pltpu.VMEM((WIN,), jnp.float32),
        pltpu.VMEM((WIN,), jnp.float32),
        pltpu.SemaphoreType.DMA,
        pltpu.SemaphoreType.DMA,
        pltpu.SemaphoreType.DMA,
        pltpu.SemaphoreType.DMA,
        pltpu.SemaphoreType.DMA,
        pltpu.SemaphoreType.DMA,
    ],
)
def _gather(table_hbm, idx_hbm, out_hbm, tbl_sp,
            idx_v0, idx_v1, out_v0, out_v1,
            in_s0, in_s1, g_s0, g_s1, st_s0, st_s1):
    c = lax.axis_index("c")
    s = lax.axis_index("s")
    wid = s * NC + c
    base = wid * (N2 // NW)

    idx_v = (idx_v0, idx_v1)
    out_v = (out_v0, out_v1)
    in_s = (in_s0, in_s1)

    in_h = [None, None]
    st_h = [None, None]
    in_h[0] = pltpu.async_copy(idx_hbm.at[pl.ds(base, WIN)], idx_v[0], in_s[0])

    seg = TSTG // NS
    pltpu.sync_copy(table_hbm.at[pl.ds(s * seg, seg)], tbl_sp.at[pl.ds(s * seg, seg)])
    plsc.subcore_barrier()

    for w in range(NWIN):
        b = w & 1
        if w + 1 < NWIN:
            off = base + (w + 1) * WIN
            in_h[1 - b] = pltpu.async_copy(
                idx_hbm.at[pl.ds(off, WIN)], idx_v[1 - b], in_s[1 - b]
            )
        in_h[b].wait()
        if w >= 2:
            st_h[b].wait()
        pltpu.async_copy(tbl_sp.at[idx_v[b]], out_v[b], (g_s0, g_s1)[b]).wait()
        st_h[b] = pltpu.async_copy(
            out_v[b], out_hbm.at[pl.ds(base + w * WIN, WIN)], (st_s0, st_s1)[b]
        )
    st_h[0].wait()
    st_h[1].wait()


def kernel(inputs, item_rating_logits):
    pad = jnp.full((TBL - NUM_ITEMS + 1,), -1e30, jnp.float32)
    x = jnp.concatenate([item_rating_logits, pad]).reshape(TR, TC_)
    table = pl.pallas_call(
        _table_body,
        out_shape=jax.ShapeDtypeStruct((TBL,), jnp.float32),
    )(x)
    idxt = pl.pallas_call(
        _pre_body,
        grid=(GB,),
        in_specs=[pl.BlockSpec((BR, COLS), lambda i: (i, 0))],
        out_specs=pl.BlockSpec((BG, 2, 8, 128), lambda i: (i, 0, 0, 0)),
        out_shape=jax.ShapeDtypeStruct((GB * BG, 2, 8, 128), jnp.int32),
    )(inputs[0])
    outt = _gather(table, idxt.reshape(N2))
    out = pl.pallas_call(
        _post_body,
        grid=(GB,),
        in_specs=[pl.BlockSpec((BG, 2, 8, 128), lambda i: (i, 0, 0, 0))],
        out_specs=pl.BlockSpec((BR, COLS), lambda i: (i, 0)),
        out_shape=jax.ShapeDtypeStruct((ROWS, COLS), jnp.float32),
    )(outt.reshape(GB * BG, 2, 8, 128))
    return out
